# baseline (speedup 1.0000x reference)
import functools
import jax
import jax.numpy as jnp
from jax import lax
from jax.experimental import pallas as pl
from jax.experimental.pallas import tpu as pltpu
from jax.experimental.pallas import tpu_sc as plsc

NUM_C = 7
B, H, W = 16, 256, 256
N_WORKERS = 32
CHUNK_ROWS = 16
ROWS_PER_WORKER = (B * H) // N_WORKERS
N_CHUNKS = ROWS_PER_WORKER // CHUNK_ROWS
LANES = 16


def _sc_onehot_body(frame_hbm, out_hbm, in0, in1, oh0, oh1,
                    s_in0, s_in1, s_out0, s_out1):
    cid = lax.axis_index("c")
    sid = lax.axis_index("s")
    w = sid * 2 + cid
    b = w // 2
    r_base = (w % 2) * ROWS_PER_WORKER

    bufs = [(in0, s_in0, oh0, s_out0), (in1, s_in1, oh1, s_out1)]

    def in_copy(k, ib, isem):
        r0 = r_base + k * CHUNK_ROWS
        return pltpu.make_async_copy(
            frame_hbm.at[b, pl.ds(r0, CHUNK_ROWS)], ib, isem)

    def out_copy(k, c, ob, osem):
        r0 = r_base + k * CHUNK_ROWS
        return pltpu.make_async_copy(
            ob.at[c], out_hbm.at[b, c, 2, pl.ds(r0, CHUNK_ROWS)], osem)

    in_copy(0, in0, s_in0).start()
    for k in range(N_CHUNKS):
        ib, isem, ob, osem = bufs[k % 2]
        in_copy(k, ib, isem).wait()
        if k + 1 < N_CHUNKS:
            nib, nisem, _, _ = bufs[(k + 1) % 2]
            in_copy(k + 1, nib, nisem).start()
        if k >= 2:
            for c in range(NUM_C):
                out_copy(k - 2, c, ob, osem).wait()

        def row_body(i, carry):
            for kk in range(W // LANES):
                f = ib[i, pl.ds(kk * LANES, LANES)]
                for c in range(NUM_C):
                    ob[c, i, pl.ds(kk * LANES, LANES)] = jnp.where(
                        f == c, jnp.float32(1.0), jnp.float32(0.0))
            return carry
        lax.fori_loop(0, CHUNK_ROWS, row_body, 0)

        for c in range(NUM_C):
            out_copy(k, c, ob, osem).start()

    for k in (N_CHUNKS - 2, N_CHUNKS - 1):
        ib, isem, ob, osem = bufs[k % 2]
        for c in range(NUM_C):
            out_copy(k, c, ob, osem).wait()


def _sc_onehot():
    mesh = plsc.VectorSubcoreMesh(
        core_axis_name="c", subcore_axis_name="s", num_cores=2, num_subcores=16)
    return pl.kernel(
        _sc_onehot_body,
        out_type=(),
        mesh=mesh,
        scratch_types=[
            pltpu.VMEM((CHUNK_ROWS, W), jnp.int32),
            pltpu.VMEM((CHUNK_ROWS, W), jnp.int32),
            pltpu.VMEM((NUM_C, CHUNK_ROWS, W), jnp.float32),
            pltpu.VMEM((NUM_C, CHUNK_ROWS, W), jnp.float32),
            pltpu.SemaphoreType.DMA,
            pltpu.SemaphoreType.DMA,
            pltpu.SemaphoreType.DMA,
            pltpu.SemaphoreType.DMA,
        ],
    )


def _tc_fill_body(_, out_ref):
    rows = jax.lax.broadcasted_iota(jnp.int32, (H, W), 0).astype(jnp.float32)
    cols = jax.lax.broadcasted_iota(jnp.int32, (H, W), 1).astype(jnp.float32)
    for bb in range(B):
        out_ref[bb, 0, 0] = rows
        out_ref[bb, 0, 1] = cols


def _tc_fill(buf5):
    return pl.pallas_call(
        _tc_fill_body,
        grid=(NUM_C,),
        in_specs=[pl.BlockSpec(memory_space=pl.ANY)],
        out_specs=pl.BlockSpec((B, 1, 2, H, W), lambda c: (0, c, 0, 0, 0)),
        out_shape=jax.ShapeDtypeStruct((B, NUM_C, 3, H, W), jnp.float32),
    )(buf5)


def kernel(frame, embed_weights):
    del embed_weights
    buf = jax.new_ref(jnp.zeros((B, NUM_C, 3, H, W), jnp.float32))
    _sc_onehot()(frame, buf)
    return buf[...].reshape(B, 3 * NUM_C, H, W)


# speedup vs baseline: 1.0946x; 1.0946x over previous
import functools
import jax
import jax.numpy as jnp
from jax import lax
from jax.experimental import pallas as pl
from jax.experimental.pallas import tpu as pltpu
from jax.experimental.pallas import tpu_sc as plsc

NUM_C = 7
B, H, W = 16, 256, 256
N_WORKERS = 32
CHUNK_ROWS = 16
ROWS_PER_WORKER = (B * H) // N_WORKERS
N_CHUNKS = ROWS_PER_WORKER // CHUNK_ROWS
LANES = 16


def _sc_onehot_body(frame_hbm, out_hbm, in0, in1, oh0, oh1,
                    s_in0, s_in1, s_out0, s_out1):
    cid = lax.axis_index("c")
    sid = lax.axis_index("s")
    w = sid * 2 + cid
    b = w // 2
    r_base = (w % 2) * ROWS_PER_WORKER

    bufs = [(in0, s_in0, oh0, s_out0), (in1, s_in1, oh1, s_out1)]

    def in_copy(k, ib, isem):
        r0 = r_base + k * CHUNK_ROWS
        return pltpu.make_async_copy(
            frame_hbm.at[b, pl.ds(r0, CHUNK_ROWS)], ib, isem)

    def out_copy(k, c, ob, osem):
        r0 = r_base + k * CHUNK_ROWS
        return pltpu.make_async_copy(
            ob.at[c], out_hbm.at[b, c, 2, pl.ds(r0, CHUNK_ROWS)], osem)

    in_copy(0, in0, s_in0).start()
    for k in range(N_CHUNKS):
        ib, isem, ob, osem = bufs[k % 2]
        in_copy(k, ib, isem).wait()
        if k + 1 < N_CHUNKS:
            nib, nisem, _, _ = bufs[(k + 1) % 2]
            in_copy(k + 1, nib, nisem).start()
        if k >= 2:
            for c in range(NUM_C):
                out_copy(k - 2, c, ob, osem).wait()

        def row_body(i, carry):
            for kk in range(W // LANES):
                f = ib[i, pl.ds(kk * LANES, LANES)]
                for c in range(NUM_C):
                    ob[c, i, pl.ds(kk * LANES, LANES)] = jnp.where(
                        f == c, jnp.float32(1.0), jnp.float32(0.0))
            return carry
        lax.fori_loop(0, CHUNK_ROWS, row_body, 0)

        for c in range(NUM_C):
            out_copy(k, c, ob, osem).start()

    for k in (N_CHUNKS - 2, N_CHUNKS - 1):
        ib, isem, ob, osem = bufs[k % 2]
        for c in range(NUM_C):
            out_copy(k, c, ob, osem).wait()


def _sc_onehot():
    mesh = plsc.VectorSubcoreMesh(
        core_axis_name="c", subcore_axis_name="s", num_cores=2, num_subcores=16)
    return pl.kernel(
        _sc_onehot_body,
        out_type=(),
        mesh=mesh,
        scratch_types=[
            pltpu.VMEM((CHUNK_ROWS, W), jnp.int32),
            pltpu.VMEM((CHUNK_ROWS, W), jnp.int32),
            pltpu.VMEM((NUM_C, CHUNK_ROWS, W), jnp.float32),
            pltpu.VMEM((NUM_C, CHUNK_ROWS, W), jnp.float32),
            pltpu.SemaphoreType.DMA,
            pltpu.SemaphoreType.DMA,
            pltpu.SemaphoreType.DMA,
            pltpu.SemaphoreType.DMA,
        ],
    )


def _tc_fill_body(buf_hbm, tmpl, sem):
    rows = jax.lax.broadcasted_iota(jnp.int32, (H, W), 0).astype(jnp.float32)
    cols = jax.lax.broadcasted_iota(jnp.int32, (H, W), 1).astype(jnp.float32)
    tmpl[0] = rows
    tmpl[1] = cols

    def copy(bb, c):
        return pltpu.make_async_copy(tmpl, buf_hbm.at[bb, c, pl.ds(0, 2)], sem)

    for bb in range(B):
        for c in range(NUM_C):
            copy(bb, c).start()
        if bb > 0:
            for c in range(NUM_C):
                copy(bb - 1, c).wait()
    for c in range(NUM_C):
        copy(B - 1, c).wait()


def _tc_fill():
    mesh = pltpu.create_tensorcore_mesh("t")
    return pl.kernel(
        _tc_fill_body,
        out_type=(),
        mesh=mesh,
        scratch_types=[
            pltpu.VMEM((2, H, W), jnp.float32),
            pltpu.SemaphoreType.DMA,
        ],
    )


def kernel(frame, embed_weights):
    del embed_weights
    buf = jax.empty_ref(jax.ShapeDtypeStruct((B, NUM_C, 3, H, W), jnp.float32))
    _sc_onehot()(frame, buf)
    _tc_fill()(buf)
    return buf[...].reshape(B, 3 * NUM_C, H, W)


# speedup vs baseline: 1.1633x; 1.0627x over previous
import functools
import jax
import jax.numpy as jnp
from jax import lax
from jax.experimental import pallas as pl
from jax.experimental.pallas import tpu as pltpu
from jax.experimental.pallas import tpu_sc as plsc

NUM_C = 7
B, H, W = 16, 256, 256
N_WORKERS = 32
CHUNK_ROWS = 16
ROWS_PER_WORKER = (B * H) // N_WORKERS
N_CHUNKS = ROWS_PER_WORKER // CHUNK_ROWS
LANES = 16


def _sc_onehot_body(frame_hbm, out_hbm, in0, in1, oh0, oh1,
                    s_in0, s_in1, s_out0, s_out1):
    cid = lax.axis_index("c")
    sid = lax.axis_index("s")
    w = sid * 2 + cid
    b = w // 2
    r_base = (w % 2) * ROWS_PER_WORKER

    bufs = [(in0, s_in0, oh0, s_out0), (in1, s_in1, oh1, s_out1)]

    def in_copy(k, ib, isem):
        r0 = r_base + k * CHUNK_ROWS
        return pltpu.make_async_copy(
            frame_hbm.at[b, pl.ds(r0, CHUNK_ROWS)], ib, isem)

    def out_copy(k, c, ob, osem):
        r0 = r_base + k * CHUNK_ROWS
        return pltpu.make_async_copy(
            ob.at[c], out_hbm.at[b, c, 2, pl.ds(r0, CHUNK_ROWS)], osem)

    in_copy(0, in0, s_in0).start()
    for k in range(N_CHUNKS):
        ib, isem, ob, osem = bufs[k % 2]
        in_copy(k, ib, isem).wait()
        if k + 1 < N_CHUNKS:
            nib, nisem, _, _ = bufs[(k + 1) % 2]
            in_copy(k + 1, nib, nisem).start()
        if k >= 2:
            for c in range(NUM_C):
                out_copy(k - 2, c, ob, osem).wait()

        def row_body(i, carry):
            for kk in range(W // LANES):
                f = ib[i, pl.ds(kk * LANES, LANES)]
                for c in range(NUM_C):
                    ob[c, i, pl.ds(kk * LANES, LANES)] = jnp.where(
                        f == c, jnp.float32(1.0), jnp.float32(0.0))
            return carry
        lax.fori_loop(0, CHUNK_ROWS, row_body, 0)

        for c in range(NUM_C):
            out_copy(k, c, ob, osem).start()

    for k in (N_CHUNKS - 2, N_CHUNKS - 1):
        ib, isem, ob, osem = bufs[k % 2]
        for c in range(NUM_C):
            out_copy(k, c, ob, osem).wait()


def _sc_onehot():
    mesh = plsc.VectorSubcoreMesh(
        core_axis_name="c", subcore_axis_name="s", num_cores=2, num_subcores=16)
    return pl.kernel(
        _sc_onehot_body,
        out_type=(),
        mesh=mesh,
        scratch_types=[
            pltpu.VMEM((CHUNK_ROWS, W), jnp.int32),
            pltpu.VMEM((CHUNK_ROWS, W), jnp.int32),
            pltpu.VMEM((NUM_C, CHUNK_ROWS, W), jnp.float32),
            pltpu.VMEM((NUM_C, CHUNK_ROWS, W), jnp.float32),
            pltpu.SemaphoreType.DMA,
            pltpu.SemaphoreType.DMA,
            pltpu.SemaphoreType.DMA,
            pltpu.SemaphoreType.DMA,
        ],
    )


def _tc_fill_body(_, out_ref):
    rows = jax.lax.broadcasted_iota(jnp.int32, (H, W), 0).astype(jnp.float32)
    cols = jax.lax.broadcasted_iota(jnp.int32, (H, W), 1).astype(jnp.float32)
    for bb in range(B):
        out_ref[bb, 0, 0] = rows
        out_ref[bb, 0, 1] = cols


def _tc_fill(buf5):
    return pl.pallas_call(
        _tc_fill_body,
        grid=(NUM_C,),
        in_specs=[pl.BlockSpec(memory_space=pl.ANY)],
        out_specs=pl.BlockSpec((B, 1, 2, H, W), lambda c: (0, c, 0, 0, 0)),
        out_shape=jax.ShapeDtypeStruct((B, NUM_C, 3, H, W), jnp.float32),
    )(buf5)


def kernel(frame, embed_weights):
    del embed_weights
    filled5 = _tc_fill(frame)
    buf = jax.new_ref(filled5)
    _sc_onehot()(frame, buf)
    return buf[...].reshape(B, 3 * NUM_C, H, W)


# speedup vs baseline: 1.1849x; 1.0186x over previous
import functools
import jax
import jax.numpy as jnp
from jax import lax
from jax.experimental import pallas as pl
from jax.experimental.pallas import tpu as pltpu
from jax.experimental.pallas import tpu_sc as plsc

NUM_C = 7
B, H, W = 16, 256, 256
N_WORKERS = 32
CHUNK_ROWS = 16
ROWS_PER_WORKER = (B * H) // N_WORKERS
N_CHUNKS = ROWS_PER_WORKER // CHUNK_ROWS
LANES = 16


def _sc_onehot_body(frame_hbm, out_hbm, in0, in1, oh0, oh1,
                    s_in0, s_in1, s_out0, s_out1):
    cid = lax.axis_index("c")
    sid = lax.axis_index("s")
    w = sid * 2 + cid
    b = w // 2
    r_base = (w % 2) * ROWS_PER_WORKER

    bufs = [(in0, s_in0, oh0, s_out0), (in1, s_in1, oh1, s_out1)]

    def in_copy(k, ib, isem):
        r0 = r_base + k * CHUNK_ROWS
        return pltpu.make_async_copy(
            frame_hbm.at[b, pl.ds(r0, CHUNK_ROWS)], ib, isem)

    def out_copy(k, c, ob, osem):
        r0 = r_base + k * CHUNK_ROWS
        return pltpu.make_async_copy(
            ob.at[c], out_hbm.at[b, c, 2, pl.ds(r0, CHUNK_ROWS)], osem)

    in_copy(0, in0, s_in0).start()
    for k in range(N_CHUNKS):
        ib, isem, ob, osem = bufs[k % 2]
        in_copy(k, ib, isem).wait()
        if k + 1 < N_CHUNKS:
            nib, nisem, _, _ = bufs[(k + 1) % 2]
            in_copy(k + 1, nib, nisem).start()
        if k >= 2:
            for c in range(NUM_C):
                out_copy(k - 2, c, ob, osem).wait()

        def row_body(i, carry):
            for kk in range(W // LANES):
                f = ib[i, pl.ds(kk * LANES, LANES)]
                for c in range(NUM_C):
                    ob[c, i, pl.ds(kk * LANES, LANES)] = jnp.where(
                        f == c, jnp.float32(1.0), jnp.float32(0.0))
            return carry
        lax.fori_loop(0, CHUNK_ROWS, row_body, 0)

        for c in range(NUM_C):
            out_copy(k, c, ob, osem).start()

    for k in (N_CHUNKS - 2, N_CHUNKS - 1):
        ib, isem, ob, osem = bufs[k % 2]
        for c in range(NUM_C):
            out_copy(k, c, ob, osem).wait()


def _sc_onehot():
    mesh = plsc.VectorSubcoreMesh(
        core_axis_name="c", subcore_axis_name="s", num_cores=2, num_subcores=16)
    return pl.kernel(
        _sc_onehot_body,
        out_type=(),
        mesh=mesh,
        scratch_types=[
            pltpu.VMEM((CHUNK_ROWS, W), jnp.int32),
            pltpu.VMEM((CHUNK_ROWS, W), jnp.int32),
            pltpu.VMEM((NUM_C, CHUNK_ROWS, W), jnp.float32),
            pltpu.VMEM((NUM_C, CHUNK_ROWS, W), jnp.float32),
            pltpu.SemaphoreType.DMA,
            pltpu.SemaphoreType.DMA,
            pltpu.SemaphoreType.DMA,
            pltpu.SemaphoreType.DMA,
        ],
    )


def _tc_fill_body(buf_hbm, tmpl, sem):
    rows = jax.lax.broadcasted_iota(jnp.int32, (H, W), 0).astype(jnp.float32)
    cols = jax.lax.broadcasted_iota(jnp.int32, (H, W), 1).astype(jnp.float32)
    tmpl[0] = rows
    tmpl[1] = cols

    def copy(bb, c):
        return pltpu.make_async_copy(tmpl, buf_hbm.at[bb, c, pl.ds(0, 2)], sem)

    for bb in range(B):
        for c in range(NUM_C):
            copy(bb, c).start()
        if bb > 0:
            for c in range(NUM_C):
                copy(bb - 1, c).wait()
    for c in range(NUM_C):
        copy(B - 1, c).wait()


def _tc_fill():
    mesh = pltpu.create_tensorcore_mesh("t")
    return pl.kernel(
        _tc_fill_body,
        out_type=(),
        mesh=mesh,
        scratch_types=[
            pltpu.VMEM((2, H, W), jnp.float32),
            pltpu.SemaphoreType.DMA,
        ],
    )


def kernel(frame, embed_weights):
    del embed_weights
    buf = jax.empty_ref(jax.ShapeDtypeStruct((B, NUM_C, 3, H, W), jnp.float32))
    _tc_fill()(buf)
    _sc_onehot()(frame, buf)
    return buf[...].reshape(B, 3 * NUM_C, H, W)
